# Initial kernel scaffold; baseline (speedup 1.0000x reference)
import jax
import jax.numpy as jnp
from jax.experimental import pallas as pl


def kernel(X, W1, b1, mem_keys, mem_values, Wq, Wo, ln1_g, ln1_b, W2, b2, ln2_g, ln2_b):
    raise NotImplementedError("write your pallas kernel here")


# speedup vs baseline: 3.0009x; 3.0009x over previous
import functools

import jax
import jax.numpy as jnp
from jax.experimental import pallas as pl

_B_BLK = 256


def _fused_kernel(x_ref, w1_ref, b1_ref, mv_ref, wo_ref, g1_ref, be1_ref,
                  w2_ref, b2_ref, g2_ref, be2_ref, out_ref):
    h = jax.lax.dot_general(
        x_ref[...], w1_ref[...],
        dimension_numbers=(((1,), (1,)), ((), ())),
        preferred_element_type=jnp.float32)
    h = jnp.maximum(h + b1_ref[...], 0.0)

    kk = mv_ref.shape[0]
    vmean = jnp.sum(mv_ref[...], axis=0, keepdims=True) / kk
    hh = wo_ref.shape[1] // mv_ref.shape[1]
    c_hv = jnp.concatenate([vmean] * hh, axis=1)
    c = jax.lax.dot_general(
        c_hv, wo_ref[...],
        dimension_numbers=(((1,), (1,)), ((), ())),
        preferred_element_type=jnp.float32)

    mem_out = h + c

    mean1 = jnp.mean(mem_out, axis=1, keepdims=True)
    cen1 = mem_out - mean1
    var1 = jnp.mean(cen1 * cen1, axis=1, keepdims=True)
    x = cen1 / jnp.sqrt(var1 + 1e-5) * g1_ref[...] + be1_ref[...]

    x2 = jax.lax.dot_general(
        x, w2_ref[...],
        dimension_numbers=(((1,), (1,)), ((), ())),
        preferred_element_type=jnp.float32)
    x2 = x2 + b2_ref[...]

    mean2 = jnp.mean(x2, axis=1, keepdims=True)
    cen2 = x2 - mean2
    var2 = jnp.mean(cen2 * cen2, axis=1, keepdims=True)
    x2 = cen2 / jnp.sqrt(var2 + 1e-5) * g2_ref[...] + be2_ref[...]
    out_ref[...] = jax.nn.sigmoid(x2)


@functools.partial(jax.jit, static_argnames=())
def kernel(X, W1, b1, mem_keys, mem_values, Wq, Wo, ln1_g, ln1_b,
           W2, b2, ln2_g, ln2_b):
    del mem_keys, Wq
    B, D = X.shape
    O = W2.shape[0]
    grid = (B // _B_BLK,)

    def rows(i):
        return (i, 0)

    def whole(i):
        return (0, 0)

    return pl.pallas_call(
        _fused_kernel,
        grid=grid,
        in_specs=[
            pl.BlockSpec((_B_BLK, D), rows),
            pl.BlockSpec((D, D), whole),
            pl.BlockSpec((1, D), whole),
            pl.BlockSpec(mem_values.shape, whole),
            pl.BlockSpec(Wo.shape, whole),
            pl.BlockSpec((1, D), whole),
            pl.BlockSpec((1, D), whole),
            pl.BlockSpec(W2.shape, whole),
            pl.BlockSpec((1, O), whole),
            pl.BlockSpec((1, O), whole),
            pl.BlockSpec((1, O), whole),
        ],
        out_specs=pl.BlockSpec((_B_BLK, O), rows),
        out_shape=jax.ShapeDtypeStruct((B, O), jnp.float32),
    )(X, W1, b1.reshape(1, D), mem_values, Wo,
      ln1_g.reshape(1, D), ln1_b.reshape(1, D), W2,
      b2.reshape(1, O), ln2_g.reshape(1, O), ln2_b.reshape(1, O))
